# baseline (speedup 1.0000x reference)
import jax
import jax.numpy as jnp
from jax import lax
from jax.experimental import pallas as pl
from jax.experimental.pallas import tpu as pltpu
from jax.experimental.pallas import tpu_sc as plsc

_NC = 2
_NS = 16
_NW = _NC * _NS
_T = 128
_C = 8
_D = 32
_S = 4
_PP = _S * _D + 1


def _splat(v):
    return jnp.full((16,), v, dtype=jnp.int32)


def _sc_body(piv_hbm, ctx_hbm, g_hbm, km_hbm, ws_hbm, wc_hbm, out_hbm,
             pivall, cidxall, g0, g1, km_v,
             pv0, pv1, ctx0, ctx1, out0, out1, pvp,
             semi0, semi1, semg0, semg1, semo0, semo1):
    gv = (g0, g1)
    pv = (pv0, pv1)
    ctxv = (ctx0, ctx1)
    outv = (out0, out1)
    semi = (semi0, semi1)
    semg = (semg0, semg1)
    semo = (semo0, semo1)

    wid = lax.axis_index("s") * _NC + lax.axis_index("c")
    tok_per_w = out_hbm.shape[0] // _NW
    n_chunks = tok_per_w // _T
    pltpu.sync_copy(km_hbm, km_v)
    pltpu.sync_copy(piv_hbm.at[pl.ds(pl.multiple_of(wid * tok_per_w, _T),
                                     tok_per_w)], pivall)
    pltpu.sync_copy(ctx_hbm.at[pl.ds(pl.multiple_of(wid * tok_per_w * _C, _T),
                                     tok_per_w * _C)], cidxall)
    kvec = km_v[...]
    iota = lax.iota(jnp.int32, 16)

    def tokbase(ch):
        return pl.multiple_of(wid * tok_per_w + ch * _T, 16)

    def g_copy(ch, b):
        tb = tokbase(ch)
        return pltpu.make_async_copy(g_hbm.at[pl.ds(tb * _S, _T * _S)],
                                     gv[b], semi[b])

    def gather_copies(ch, b):
        poff = pl.multiple_of(ch * _T, _T)
        cps = [(pltpu.make_async_copy(
            ws_hbm.at[pivall.at[pl.ds(poff, _T)]], pv[b], semg[b]), False)]
        for c in range(_C):
            coff = pl.multiple_of(ch * _T * _C + c * _T, _T)
            cps.append((pltpu.make_async_copy(
                wc_hbm.at[cidxall.at[pl.ds(coff, _T)]],
                ctxv[b], semg[b]), True))
        return cps

    def start_gathers(ch, b):
        for cp, add in gather_copies(ch, b):
            cp.start(add=add)

    def wait_gathers(ch, b):
        for cp, _ in gather_copies(ch, b):
            cp.wait()

    def out_copy(ch, b):
        tb = tokbase(ch)
        return pltpu.make_async_copy(
            outv[b], out_hbm.at[pl.ds(tb, _T)], semo[b])

    def compute(b):
        g_b = gv[b]
        pv_b = pv[b]
        ctx_b = ctxv[b]
        out_b = outv[b]

        def group(g16, inner_carry):
            rowv = iota + g16 * 16

            def repack(k16, rcarry):
                for j in range(16):
                    evec = (iota + (k16 * 16 + j)) & (_S * _D - 1)
                    x = plsc.load_gather(pv_b, [rowv, evec])
                    plsc.store_scatter(pvp, [rowv, evec], x)
                return rcarry

            lax.fori_loop(0, _S * _D // 16, repack, 0)

            def prodstep(k4, prod):
                dvec = (iota + k4) & (_D - 1)
                acc = plsc.load_gather(ctx_b, [rowv, dvec])
                plsc.store_scatter(ctx_b, [rowv, dvec],
                                   jnp.zeros((16,), jnp.float32))
                col4 = dvec * _S
                return tuple(
                    prod[s] + (acc * kvec) * plsc.load_gather(
                        pvp, [rowv, col4 + s])
                    for s in range(_S))

            zero = jnp.zeros((16,), jnp.float32)
            prod = lax.fori_loop(0, _D, prodstep, (zero,) * _S)

            gbase = rowv * _S
            y = [prod[s] - plsc.load_gather(g_b, [gbase + s])
                 for s in range(_S)]
            mx = jnp.maximum(jnp.maximum(y[0], y[1]), jnp.maximum(y[2], y[3]))
            e = [jnp.exp(y[s] - mx) for s in range(_S)]
            den = (e[0] + e[1]) + (e[2] + e[3])
            att = [e[s] / den for s in range(_S)]

            def outstep(k4, ocarry):
                dvec = (iota + k4) & (_D - 1)
                col4 = dvec * _S
                o = att[0] * plsc.load_gather(pvp, [rowv, col4])
                for s in range(1, _S):
                    o = o + att[s] * plsc.load_gather(pvp, [rowv, col4 + s])
                plsc.store_scatter(out_b, [rowv, dvec], o)
                return ocarry

            lax.fori_loop(0, _D, outstep, 0)
            return inner_carry

        lax.fori_loop(0, _T // 16, group, 0)

    def zinit(t, carry):
        z = jnp.zeros((16,), jnp.float32)
        for buf in (ctx0, ctx1):
            buf[t, pl.ds(0, 16)] = z
            buf[t, pl.ds(16, 16)] = z
        return carry

    lax.fori_loop(0, _T, zinit, 0)

    start_gathers(0, 0)
    g_copy(0, 0).start()
    start_gathers(1, 1)
    g_copy(1, 1).start()

    def step(i, carry):
        for b in (0, 1):
            ch = i * 2 + b

            wait_gathers(ch, b)

            @pl.when(ch >= 2)
            def _():
                out_copy(ch - 2, b).wait()

            g_copy(ch, b).wait()
            compute(b)
            out_copy(ch, b).start()

            @pl.when(ch + 2 < n_chunks)
            def _():
                start_gathers(ch + 2, b)
                g_copy(ch + 2, b).start()
        return carry

    lax.fori_loop(0, n_chunks // 2, step, 0)
    out_copy(n_chunks - 2, 0).wait()
    out_copy(n_chunks - 1, 1).wait()


def kernel(pivots, contexts, W_sense, W_ctx, tau, scale):
    Bp, Lp = pivots.shape
    sz = Bp * Lp
    piv = pivots.reshape(sz).astype(jnp.int32)
    ctxf = contexts.astype(jnp.int32).reshape(sz * _C)
    U = jax.random.uniform(jax.random.key(42), (sz, _S), dtype=jnp.float32)
    g2 = ((scale / tau) * jnp.log(-jnp.log(U + 1e-20) + 1e-20)).reshape(-1)
    g2 = jnp.asarray(g2, jnp.float32)
    km = jnp.full((16,), 1.0, jnp.float32) / (_C * tau)

    mesh = plsc.VectorSubcoreMesh(core_axis_name="c", subcore_axis_name="s")
    out = pl.kernel(
        _sc_body,
        out_type=jax.ShapeDtypeStruct((sz, _D), jnp.float32),
        mesh=mesh,
        compiler_params=pltpu.CompilerParams(needs_layout_passes=False,
                                             use_tc_tiling_on_sc=False),
        scratch_types=[
            pltpu.VMEM((sz // _NW,), jnp.int32),
            pltpu.VMEM((sz // _NW * _C,), jnp.int32),
            pltpu.VMEM((_T * _S,), jnp.float32),
            pltpu.VMEM((_T * _S,), jnp.float32),
            pltpu.VMEM((16,), jnp.float32),
            pltpu.VMEM((_T, _S * _D), jnp.float32),
            pltpu.VMEM((_T, _S * _D), jnp.float32),
            pltpu.VMEM((_T, _D), jnp.float32),
            pltpu.VMEM((_T, _D), jnp.float32),
            pltpu.VMEM((_T, _D), jnp.float32),
            pltpu.VMEM((_T, _D), jnp.float32),
            pltpu.VMEM((_T, _PP), jnp.float32),
            pltpu.SemaphoreType.DMA,
            pltpu.SemaphoreType.DMA,
            pltpu.SemaphoreType.DMA,
            pltpu.SemaphoreType.DMA,
            pltpu.SemaphoreType.DMA,
            pltpu.SemaphoreType.DMA,
        ],
    )(piv, ctxf, g2, km, W_sense, W_ctx)
    return out.reshape(Bp, Lp, _D)
